# Initial kernel scaffold; baseline (speedup 1.0000x reference)
import jax
import jax.numpy as jnp
from jax.experimental import pallas as pl


def kernel(output, target):
    raise NotImplementedError("write your pallas kernel here")


# speedup vs baseline: 16.5027x; 16.5027x over previous
import functools

import jax
import jax.numpy as jnp
from jax.experimental import pallas as pl
from jax.experimental.pallas import tpu as pltpu

_B = 16
_N = 512 * 512
_K = int(0.15 * _N)


def _topk_mean_body(o_ref, t_ref, out_ref, loss_ref, bits_ref):
    i = pl.program_id(0)
    o = o_ref[0]
    t = t_ref[0]
    loss = jnp.maximum(o, 0.0) - o * t + jnp.log1p(jnp.exp(-jnp.abs(o)))
    loss_ref[...] = loss
    bits_ref[...] = jax.lax.bitcast_convert_type(loss, jnp.int32)

    def step(j, prefix):
        b = 30 - j
        cand = prefix | jax.lax.shift_left(jnp.int32(1), b)
        cnt = jnp.sum((bits_ref[...] >= cand).astype(jnp.int32))
        return jnp.where(cnt >= _K, cand, prefix)

    prefix = jax.lax.fori_loop(0, 31, step, jnp.int32(0))

    bits = bits_ref[...]
    gt = bits > prefix
    cnt_gt = jnp.sum(gt.astype(jnp.int32))
    sum_gt = jnp.sum(jnp.where(gt, loss_ref[...], 0.0))
    vk = jax.lax.bitcast_convert_type(prefix, jnp.float32)
    contrib = sum_gt + (_K - cnt_gt).astype(jnp.float32) * vk

    @pl.when(i == 0)
    def _init():
        out_ref[...] = jnp.zeros((1, 1), jnp.float32)

    out_ref[...] = out_ref[...] + jnp.full((1, 1), contrib)


@functools.partial(jax.jit, static_argnames=("interpret",))
def _topk_mean(o3, t3, interpret=False):
    acc = pl.pallas_call(
        _topk_mean_body,
        grid=(_B,),
        in_specs=[
            pl.BlockSpec((1, 512, 512), lambda i: (i, 0, 0)),
            pl.BlockSpec((1, 512, 512), lambda i: (i, 0, 0)),
        ],
        out_specs=pl.BlockSpec((1, 1), lambda i: (0, 0)),
        out_shape=jax.ShapeDtypeStruct((1, 1), jnp.float32),
        scratch_shapes=[
            pltpu.VMEM((512, 512), jnp.float32),
            pltpu.VMEM((512, 512), jnp.int32),
        ],
        interpret=interpret,
    )(o3, t3)
    return acc[0, 0] / jnp.float32(_B * _K)


def kernel(output, target):
    o3 = output.reshape(_B, 512, 512)
    t3 = target.reshape(_B, 512, 512)
    return _topk_mean(o3, t3)


# speedup vs baseline: 40.8392x; 2.4747x over previous
import functools

import jax
import jax.numpy as jnp
from jax.experimental import pallas as pl
from jax.experimental.pallas import tpu as pltpu

_B = 16
_N = 512 * 512
_K = int(0.15 * _N)
_RB = 8


def _topk_mean_body(o_ref, t_ref, out_ref, bits_ref):
    i = pl.program_id(0)
    o = o_ref[...]
    t = t_ref[...]
    loss = jnp.maximum(o, 0.0) - o * t + jnp.log1p(jnp.exp(-jnp.abs(o)))
    bits_ref[...] = jax.lax.bitcast_convert_type(loss, jnp.int32)

    def step(j, prefix):
        b = 30 - j
        cand = prefix | jax.lax.shift_left(jnp.int32(1), b)
        cnt = jnp.sum((bits_ref[...] >= cand).astype(jnp.int32), axis=(1, 2))
        return jnp.where((cnt >= _K).reshape(_RB, 1, 1), cand, prefix)

    prefix = jax.lax.fori_loop(0, 31, step, jnp.zeros((_RB, 1, 1), jnp.int32))

    bits = bits_ref[...]
    gt = bits > prefix
    cnt_gt = jnp.sum(gt.astype(jnp.int32), axis=(1, 2))
    lossv = jax.lax.bitcast_convert_type(bits, jnp.float32)
    sum_gt = jnp.sum(jnp.where(gt, lossv, 0.0), axis=(1, 2))
    vk = jax.lax.bitcast_convert_type(prefix.reshape(_RB), jnp.float32)
    contrib = jnp.sum(sum_gt + (_K - cnt_gt).astype(jnp.float32) * vk)

    @pl.when(i == 0)
    def _init():
        out_ref[...] = jnp.zeros((1, 1), jnp.float32)

    out_ref[...] = out_ref[...] + jnp.full((1, 1), contrib)


@functools.partial(jax.jit, static_argnames=("interpret",))
def _topk_mean(o3, t3, interpret=False):
    acc = pl.pallas_call(
        _topk_mean_body,
        grid=(_B // _RB,),
        in_specs=[
            pl.BlockSpec((_RB, 512, 512), lambda i: (i, 0, 0)),
            pl.BlockSpec((_RB, 512, 512), lambda i: (i, 0, 0)),
        ],
        out_specs=pl.BlockSpec((1, 1), lambda i: (0, 0)),
        out_shape=jax.ShapeDtypeStruct((1, 1), jnp.float32),
        scratch_shapes=[
            pltpu.VMEM((_RB, 512, 512), jnp.int32),
        ],
        interpret=interpret,
    )(o3, t3)
    return acc[0, 0] / jnp.float32(_B * _K)


def kernel(output, target):
    o3 = output.reshape(_B, 512, 512)
    t3 = target.reshape(_B, 512, 512)
    return _topk_mean(o3, t3)
